# baseline (speedup 1.0000x reference)
import numpy as np
import jax
import jax.numpy as jnp
from jax.experimental import pallas as pl
from jax.experimental.pallas import tpu as pltpu

_B, _S, _F, _C = 8, 32, 3, 64
_H = 256
_NS = 250
_G = _B * _C
_N = _G * _S
_GB = 64
_R = _GB * _S
_CH = 128
_NCH = _R // _CH


def _make_bd():
    dinv = 1.0 / np.sqrt(np.arange(1, _S + 1, dtype=np.float64))
    m = np.tril(np.outer(dinv, dinv))
    bd = np.zeros((_CH, _CH), np.float64)
    for t in range(_CH // _S):
        bd[t * _S:(t + 1) * _S, t * _S:(t + 1) * _S] = m
    return bd.astype(np.float32)


_BD = _make_bd()


def _body(x_ref, w1_ref, b1_ref, w2_ref, b2_ref,
          fc1w_ref, fc1b_ref, fc2w_ref, fc2b_ref, bd_ref,
          out_ref, acc_ref):
    i = pl.program_id(0)
    bd = bd_ref[...]
    mx = jnp.concatenate(
        [jnp.dot(bd, x_ref[t * _CH:(t + 1) * _CH, :],
                 preferred_element_type=jnp.float32) for t in range(_NCH)],
        axis=0)
    a = jnp.dot(mx.astype(jnp.bfloat16), w1_ref[...],
                preferred_element_type=jnp.float32)
    h1 = jnp.maximum(a + b1_ref[...], 0.0)
    p2 = jnp.dot(h1.astype(jnp.bfloat16), w2_ref[...],
                 preferred_element_type=jnp.float32)
    m2 = jnp.concatenate(
        [jnp.dot(bd, p2[t * _CH:(t + 1) * _CH, :].astype(jnp.bfloat16),
                 preferred_element_type=jnp.float32) for t in range(_NCH)],
        axis=0)
    h2 = jnp.maximum(m2 + b2_ref[...], 0.0)
    acc_ref[pl.ds(i, 1), :] = h2.sum(axis=0, keepdims=True) * (1.0 / _R)

    @pl.when(i == _B - 1)
    def _head():
        p = acc_ref[...]
        h = jnp.maximum(
            jnp.dot(p, fc1w_ref[...], preferred_element_type=jnp.float32)
            + fc1b_ref[...], 0.0)
        out_ref[...] = (
            jnp.dot(h, fc2w_ref[...], preferred_element_type=jnp.float32)
            + fc2b_ref[...])


def kernel(x, W1, b1, W2, b2, fc1_W, fc1_b, fc2_W, fc2_b):
    xt = jnp.transpose(x, (0, 3, 1, 2)).reshape(_N, _F).astype(jnp.bfloat16)
    return pl.pallas_call(
        _body,
        grid=(_B,),
        in_specs=[
            pl.BlockSpec((_R, _F), lambda i: (i, 0)),
            pl.BlockSpec((_F, _H), lambda i: (0, 0)),
            pl.BlockSpec((1, _H), lambda i: (0, 0)),
            pl.BlockSpec((_H, _H), lambda i: (0, 0)),
            pl.BlockSpec((1, _H), lambda i: (0, 0)),
            pl.BlockSpec((_H, _H), lambda i: (0, 0)),
            pl.BlockSpec((1, _H), lambda i: (0, 0)),
            pl.BlockSpec((_H, _NS), lambda i: (0, 0)),
            pl.BlockSpec((1, _NS), lambda i: (0, 0)),
            pl.BlockSpec((_CH, _CH), lambda i: (0, 0)),
        ],
        out_specs=pl.BlockSpec((_B, _NS), lambda i: (0, 0)),
        out_shape=jax.ShapeDtypeStruct((_B, _NS), jnp.float32),
        scratch_shapes=[pltpu.VMEM((_B, _H), jnp.float32)],
    )(xt, W1.astype(jnp.bfloat16), b1.reshape(1, _H),
      W2.astype(jnp.bfloat16), b2.reshape(1, _H),
      fc1_W, fc1_b.reshape(1, _H), fc2_W, fc2_b.reshape(1, _NS),
      jnp.asarray(_BD, jnp.bfloat16))


# speedup vs baseline: 1.0593x; 1.0593x over previous
import numpy as np
import jax
import jax.numpy as jnp
from jax.experimental import pallas as pl
from jax.experimental.pallas import tpu as pltpu

_B, _S, _F, _C = 8, 32, 3, 64
_H = 256
_NS = 250
_G = _B * _C
_N = _G * _S
_GB = 128
_R = _GB * _S
_CH = 128
_NCH = _R // _CH
_BPS = _GB // _C
_NSTEP = _B // _BPS


def _make_bd():
    dinv = 1.0 / np.sqrt(np.arange(1, _S + 1, dtype=np.float64))
    m = np.tril(np.outer(dinv, dinv))
    bd = np.zeros((_CH, _CH), np.float64)
    for t in range(_CH // _S):
        bd[t * _S:(t + 1) * _S, t * _S:(t + 1) * _S] = m
    return bd.astype(np.float32)


_BD = _make_bd()


def _body(x_ref, w1_ref, b1_ref, w2_ref, b2_ref,
          fc1w_ref, fc1b_ref, fc2w_ref, fc2b_ref, bd_ref,
          out_ref, acc_ref):
    i = pl.program_id(0)
    bd = bd_ref[...]
    mx = jnp.concatenate(
        [jnp.dot(bd, x_ref[t * _CH:(t + 1) * _CH, :],
                 preferred_element_type=jnp.float32) for t in range(_NCH)],
        axis=0)
    a = jnp.dot(mx.astype(jnp.bfloat16), w1_ref[...],
                preferred_element_type=jnp.float32)
    h1 = jnp.maximum(a + b1_ref[...], 0.0)
    p2 = jnp.dot(h1.astype(jnp.bfloat16), w2_ref[...],
                 preferred_element_type=jnp.float32)
    m2 = jnp.concatenate(
        [jnp.dot(bd, p2[t * _CH:(t + 1) * _CH, :].astype(jnp.bfloat16),
                 preferred_element_type=jnp.float32) for t in range(_NCH)],
        axis=0)
    h2 = jnp.maximum(m2 + b2_ref[...], 0.0)
    rows = _C * _S
    sums = jnp.concatenate(
        [h2[k * rows:(k + 1) * rows, :].sum(axis=0, keepdims=True)
         for k in range(_BPS)]
        + [jnp.zeros((8 - _BPS, _H), jnp.float32)], axis=0)
    acc_ref[pl.ds(i * 8, 8), :] = sums * (1.0 / rows)

    @pl.when(i == _NSTEP - 1)
    def _head():
        p = jnp.concatenate(
            [acc_ref[k * 8:k * 8 + _BPS, :] for k in range(_NSTEP)], axis=0)
        h = jnp.maximum(
            jnp.dot(p, fc1w_ref[...], preferred_element_type=jnp.float32)
            + fc1b_ref[...], 0.0)
        out_ref[...] = (
            jnp.dot(h, fc2w_ref[...], preferred_element_type=jnp.float32)
            + fc2b_ref[...])


def kernel(x, W1, b1, W2, b2, fc1_W, fc1_b, fc2_W, fc2_b):
    xt = jnp.transpose(x, (0, 3, 1, 2)).reshape(_N, _F).astype(jnp.bfloat16)
    return pl.pallas_call(
        _body,
        grid=(_NSTEP,),
        in_specs=[
            pl.BlockSpec((_R, _F), lambda i: (i, 0)),
            pl.BlockSpec((_F, _H), lambda i: (0, 0)),
            pl.BlockSpec((1, _H), lambda i: (0, 0)),
            pl.BlockSpec((_H, _H), lambda i: (0, 0)),
            pl.BlockSpec((1, _H), lambda i: (0, 0)),
            pl.BlockSpec((_H, _H), lambda i: (0, 0)),
            pl.BlockSpec((1, _H), lambda i: (0, 0)),
            pl.BlockSpec((_H, _NS), lambda i: (0, 0)),
            pl.BlockSpec((1, _NS), lambda i: (0, 0)),
            pl.BlockSpec((_CH, _CH), lambda i: (0, 0)),
        ],
        out_specs=pl.BlockSpec((_B, _NS), lambda i: (0, 0)),
        out_shape=jax.ShapeDtypeStruct((_B, _NS), jnp.float32),
        scratch_shapes=[pltpu.VMEM((_NSTEP * 8, _H), jnp.float32)],
    )(xt, W1.astype(jnp.bfloat16), b1.reshape(1, _H),
      W2.astype(jnp.bfloat16), b2.reshape(1, _H),
      fc1_W, fc1_b.reshape(1, _H), fc2_W, fc2_b.reshape(1, _NS),
      jnp.asarray(_BD, jnp.bfloat16))


# speedup vs baseline: 1.0606x; 1.0013x over previous
import numpy as np
import jax
import jax.numpy as jnp
from jax.experimental import pallas as pl
from jax.experimental.pallas import tpu as pltpu

_B, _S, _F, _C = 8, 32, 3, 64
_H = 256
_NS = 250
_G = _B * _C
_N = _G * _S
_GB = 128
_R = _GB * _S
_CH = 128
_NCH = _R // _CH
_BPS = _GB // _C
_NSTEP = _B // _BPS


def _make_bd():
    dinv = 1.0 / np.sqrt(np.arange(1, _S + 1, dtype=np.float64))
    m = np.tril(np.outer(dinv, dinv))
    bd = np.zeros((_CH, _CH), np.float64)
    for t in range(_CH // _S):
        bd[t * _S:(t + 1) * _S, t * _S:(t + 1) * _S] = m
    return bd.astype(np.float32)


_BD = _make_bd()


def _body(x_ref, w1_ref, b1_ref, w2_ref, b2_ref,
          fc1w_ref, fc1b_ref, fc2w_ref, fc2b_ref, bd_ref,
          out_ref, acc_ref):
    i = pl.program_id(0)
    bd = bd_ref[...]
    b1 = b1_ref[...].astype(jnp.bfloat16)
    b2 = b2_ref[...]
    mx = jnp.concatenate(
        [jnp.dot(bd, x_ref[t * _CH:(t + 1) * _CH, :],
                 preferred_element_type=jnp.float32) for t in range(_NCH)],
        axis=0)
    a = jnp.dot(mx.astype(jnp.bfloat16), w1_ref[...],
                preferred_element_type=jnp.float32)
    h1 = jnp.maximum(a + b1, 0.0).astype(jnp.bfloat16)
    p2 = jnp.dot(h1, w2_ref[...],
                 preferred_element_type=jnp.float32).astype(jnp.bfloat16)
    rows = _C * _S
    csums = []
    for k in range(_BPS):
        racc = jnp.zeros((_CH, _H), jnp.float32)
        for t in range(k * rows // _CH, (k + 1) * rows // _CH):
            m2c = jnp.dot(bd, p2[t * _CH:(t + 1) * _CH, :],
                          preferred_element_type=jnp.float32)
            racc = racc + jnp.maximum(m2c + b2, 0.0)
        csums.append(racc.sum(axis=0, keepdims=True))
    sums = jnp.concatenate(
        csums + [jnp.zeros((8 - _BPS, _H), jnp.float32)], axis=0)
    acc_ref[pl.ds(i * 8, 8), :] = sums * (1.0 / rows)

    @pl.when(i == _NSTEP - 1)
    def _head():
        p = jnp.concatenate(
            [acc_ref[k * 8:k * 8 + _BPS, :] for k in range(_NSTEP)], axis=0)
        h = jnp.maximum(
            jnp.dot(p, fc1w_ref[...], preferred_element_type=jnp.float32)
            + fc1b_ref[...], 0.0)
        out_ref[...] = (
            jnp.dot(h, fc2w_ref[...], preferred_element_type=jnp.float32)
            + fc2b_ref[...])


def kernel(x, W1, b1, W2, b2, fc1_W, fc1_b, fc2_W, fc2_b):
    xt = jnp.transpose(x, (0, 3, 1, 2)).reshape(_N, _F).astype(jnp.bfloat16)
    return pl.pallas_call(
        _body,
        grid=(_NSTEP,),
        in_specs=[
            pl.BlockSpec((_R, _F), lambda i: (i, 0)),
            pl.BlockSpec((_F, _H), lambda i: (0, 0)),
            pl.BlockSpec((1, _H), lambda i: (0, 0)),
            pl.BlockSpec((_H, _H), lambda i: (0, 0)),
            pl.BlockSpec((1, _H), lambda i: (0, 0)),
            pl.BlockSpec((_H, _H), lambda i: (0, 0)),
            pl.BlockSpec((1, _H), lambda i: (0, 0)),
            pl.BlockSpec((_H, _NS), lambda i: (0, 0)),
            pl.BlockSpec((1, _NS), lambda i: (0, 0)),
            pl.BlockSpec((_CH, _CH), lambda i: (0, 0)),
        ],
        out_specs=pl.BlockSpec((_B, _NS), lambda i: (0, 0)),
        out_shape=jax.ShapeDtypeStruct((_B, _NS), jnp.float32),
        scratch_shapes=[pltpu.VMEM((_NSTEP * 8, _H), jnp.float32)],
    )(xt, W1.astype(jnp.bfloat16), b1.reshape(1, _H),
      W2.astype(jnp.bfloat16), b2.reshape(1, _H),
      fc1_W, fc1_b.reshape(1, _H), fc2_W, fc2_b.reshape(1, _NS),
      jnp.asarray(_BD, jnp.bfloat16))


# speedup vs baseline: 1.1049x; 1.0417x over previous
import numpy as np
import jax
import jax.numpy as jnp
from jax.experimental import pallas as pl
from jax.experimental.pallas import tpu as pltpu

_B, _S, _F, _C = 8, 32, 3, 64
_H = 256
_NS = 250
_G = _B * _C
_N = _G * _S
_GB = 128
_R = _GB * _S
_CH = 128
_NCH = _R // _CH
_BPS = _GB // _C
_NSTEP = _B // _BPS


def _make_bd():
    dinv = 1.0 / np.sqrt(np.arange(1, _S + 1, dtype=np.float64))
    m = np.tril(np.outer(dinv, dinv))
    bd = np.zeros((_CH, _CH), np.float64)
    for t in range(_CH // _S):
        bd[t * _S:(t + 1) * _S, t * _S:(t + 1) * _S] = m
    return bd.astype(np.float32)


_BD = _make_bd()


def _body(x_ref, w1_ref, b1_ref, w2_ref, b2_ref,
          fc1w_ref, fc1b_ref, fc2w_ref, fc2b_ref, bd_ref,
          out_ref, acc_ref):
    i = pl.program_id(0)
    bd = bd_ref[...]
    b1 = b1_ref[...].astype(jnp.bfloat16)
    b2 = b2_ref[...]
    w1 = w1_ref[...].astype(jnp.bfloat16)
    w2 = w2_ref[...].astype(jnp.bfloat16)
    mx = jnp.concatenate(
        [jnp.dot(bd, x_ref[t * _CH:(t + 1) * _CH, :].astype(jnp.bfloat16),
                 preferred_element_type=jnp.float32) for t in range(_NCH)],
        axis=0)
    a = jnp.dot(mx.astype(jnp.bfloat16), w1,
                preferred_element_type=jnp.float32)
    h1 = jnp.maximum(a + b1, 0.0).astype(jnp.bfloat16)
    p2 = jnp.dot(h1, w2,
                 preferred_element_type=jnp.float32).astype(jnp.bfloat16)
    rows = _C * _S
    csums = []
    for k in range(_BPS):
        racc = jnp.zeros((_CH, _H), jnp.float32)
        for t in range(k * rows // _CH, (k + 1) * rows // _CH):
            m2c = jnp.dot(bd, p2[t * _CH:(t + 1) * _CH, :],
                          preferred_element_type=jnp.float32)
            racc = racc + jnp.maximum(m2c + b2, 0.0)
        csums.append(racc.sum(axis=0, keepdims=True))
    sums = jnp.concatenate(
        csums + [jnp.zeros((8 - _BPS, _H), jnp.float32)], axis=0)
    acc_ref[pl.ds(i * 8, 8), :] = sums * (1.0 / rows)

    @pl.when(i == _NSTEP - 1)
    def _head():
        p = jnp.concatenate(
            [acc_ref[k * 8:k * 8 + _BPS, :] for k in range(_NSTEP)], axis=0)
        h = jnp.maximum(
            jnp.dot(p, fc1w_ref[...], preferred_element_type=jnp.float32)
            + fc1b_ref[...], 0.0)
        out_ref[...] = (
            jnp.dot(h, fc2w_ref[...], preferred_element_type=jnp.float32)
            + fc2b_ref[...])


def kernel(x, W1, b1, W2, b2, fc1_W, fc1_b, fc2_W, fc2_b):
    xt = jnp.transpose(x, (0, 3, 1, 2)).reshape(_N, _F)
    return pl.pallas_call(
        _body,
        grid=(_NSTEP,),
        in_specs=[
            pl.BlockSpec((_R, _F), lambda i: (i, 0)),
            pl.BlockSpec((_F, _H), lambda i: (0, 0)),
            pl.BlockSpec((1, _H), lambda i: (0, 0)),
            pl.BlockSpec((_H, _H), lambda i: (0, 0)),
            pl.BlockSpec((1, _H), lambda i: (0, 0)),
            pl.BlockSpec((_H, _H), lambda i: (0, 0)),
            pl.BlockSpec((1, _H), lambda i: (0, 0)),
            pl.BlockSpec((_H, _NS), lambda i: (0, 0)),
            pl.BlockSpec((1, _NS), lambda i: (0, 0)),
            pl.BlockSpec((_CH, _CH), lambda i: (0, 0)),
        ],
        out_specs=pl.BlockSpec((_B, _NS), lambda i: (0, 0)),
        out_shape=jax.ShapeDtypeStruct((_B, _NS), jnp.float32),
        scratch_shapes=[pltpu.VMEM((_NSTEP * 8, _H), jnp.float32)],
    )(xt, W1, b1.reshape(1, _H),
      W2, b2.reshape(1, _H),
      fc1_W, fc1_b.reshape(1, _H), fc2_W, fc2_b.reshape(1, _NS),
      jnp.asarray(_BD, jnp.bfloat16))


# speedup vs baseline: 1.1132x; 1.0075x over previous
import numpy as np
import jax
import jax.numpy as jnp
from jax.experimental import pallas as pl
from jax.experimental.pallas import tpu as pltpu

_B, _S, _F, _C = 8, 32, 3, 64
_H = 256
_NS = 250
_G = _B * _C
_N = _G * _S
_GB = 256
_R = _GB * _S
_CH = 128
_NCH = _R // _CH
_BPS = _GB // _C
_NSTEP = _B // _BPS


def _make_bd():
    dinv = 1.0 / np.sqrt(np.arange(1, _S + 1, dtype=np.float64))
    m = np.tril(np.outer(dinv, dinv))
    bd = np.zeros((_CH, _CH), np.float64)
    for t in range(_CH // _S):
        bd[t * _S:(t + 1) * _S, t * _S:(t + 1) * _S] = m
    return bd.astype(np.float32)


_BD = _make_bd()


def _body(x_ref, w1_ref, b1_ref, w2_ref, b2_ref,
          fc1w_ref, fc1b_ref, fc2w_ref, fc2b_ref, bd_ref,
          out_ref, acc_ref):
    i = pl.program_id(0)
    bd = bd_ref[...]
    b1 = b1_ref[...].astype(jnp.bfloat16)
    b2 = b2_ref[...]
    w1 = w1_ref[...].astype(jnp.bfloat16)
    w2 = w2_ref[...].astype(jnp.bfloat16)
    mx = jnp.concatenate(
        [jnp.dot(bd, x_ref[t * _CH:(t + 1) * _CH, :].astype(jnp.bfloat16),
                 preferred_element_type=jnp.float32) for t in range(_NCH)],
        axis=0)
    a = jnp.dot(mx.astype(jnp.bfloat16), w1,
                preferred_element_type=jnp.float32)
    h1 = jnp.maximum(a + b1, 0.0).astype(jnp.bfloat16)
    p2 = jnp.dot(h1, w2,
                 preferred_element_type=jnp.float32).astype(jnp.bfloat16)
    rows = _C * _S
    csums = []
    for k in range(_BPS):
        racc = jnp.zeros((_CH, _H), jnp.float32)
        for t in range(k * rows // _CH, (k + 1) * rows // _CH):
            m2c = jnp.dot(bd, p2[t * _CH:(t + 1) * _CH, :],
                          preferred_element_type=jnp.float32)
            racc = racc + jnp.maximum(m2c + b2, 0.0)
        csums.append(racc.sum(axis=0, keepdims=True))
    sums = jnp.concatenate(
        csums + [jnp.zeros((8 - _BPS, _H), jnp.float32)], axis=0)
    acc_ref[pl.ds(i * 8, 8), :] = sums * (1.0 / rows)

    @pl.when(i == _NSTEP - 1)
    def _head():
        p = jnp.concatenate(
            [acc_ref[k * 8:k * 8 + _BPS, :] for k in range(_NSTEP)], axis=0)
        h = jnp.maximum(
            jnp.dot(p, fc1w_ref[...], preferred_element_type=jnp.float32)
            + fc1b_ref[...], 0.0)
        out_ref[...] = (
            jnp.dot(h, fc2w_ref[...], preferred_element_type=jnp.float32)
            + fc2b_ref[...])


def kernel(x, W1, b1, W2, b2, fc1_W, fc1_b, fc2_W, fc2_b):
    xt = jnp.transpose(x, (0, 3, 1, 2)).reshape(_N, _F)
    return pl.pallas_call(
        _body,
        grid=(_NSTEP,),
        in_specs=[
            pl.BlockSpec((_R, _F), lambda i: (i, 0)),
            pl.BlockSpec((_F, _H), lambda i: (0, 0)),
            pl.BlockSpec((1, _H), lambda i: (0, 0)),
            pl.BlockSpec((_H, _H), lambda i: (0, 0)),
            pl.BlockSpec((1, _H), lambda i: (0, 0)),
            pl.BlockSpec((_H, _H), lambda i: (0, 0)),
            pl.BlockSpec((1, _H), lambda i: (0, 0)),
            pl.BlockSpec((_H, _NS), lambda i: (0, 0)),
            pl.BlockSpec((1, _NS), lambda i: (0, 0)),
            pl.BlockSpec((_CH, _CH), lambda i: (0, 0)),
        ],
        out_specs=pl.BlockSpec((_B, _NS), lambda i: (0, 0)),
        out_shape=jax.ShapeDtypeStruct((_B, _NS), jnp.float32),
        scratch_shapes=[pltpu.VMEM((_NSTEP * 8, _H), jnp.float32)],
    )(xt, W1, b1.reshape(1, _H),
      W2, b2.reshape(1, _H),
      fc1_W, fc1_b.reshape(1, _H), fc2_W, fc2_b.reshape(1, _NS),
      jnp.asarray(_BD, jnp.bfloat16))


# speedup vs baseline: 1.1513x; 1.0342x over previous
import numpy as np
import jax
import jax.numpy as jnp
from jax.experimental import pallas as pl
from jax.experimental.pallas import tpu as pltpu

_B, _S, _F, _C = 8, 32, 3, 64
_H = 256
_NS = 250
_G = _B * _C
_N = _G * _S
_GB = 256
_R = _GB * _S
_CH = 128
_NCH = _R // _CH
_BPS = _GB // _C
_NSTEP = _B // _BPS


def _make_bd():
    dinv = 1.0 / np.sqrt(np.arange(1, _S + 1, dtype=np.float64))
    m = np.tril(np.outer(dinv, dinv))
    bd = np.zeros((_CH, _CH), np.float64)
    for t in range(_CH // _S):
        bd[t * _S:(t + 1) * _S, t * _S:(t + 1) * _S] = m
    return bd.astype(np.float32)


_BD = _make_bd()


def _body(x_ref, w1_ref, b1_ref, w2_ref, b2_ref,
          fc1w_ref, fc1b_ref, fc2w_ref, fc2b_ref, bd_ref,
          out_ref, acc_ref):
    i = pl.program_id(0)
    bd = bd_ref[...]
    b1 = b1_ref[...].astype(jnp.bfloat16)
    b2 = b2_ref[...]
    w1 = w1_ref[...].astype(jnp.bfloat16)
    w2 = w2_ref[...].astype(jnp.bfloat16)
    mx = jnp.concatenate(
        [jnp.dot(bd, x_ref[t * _CH:(t + 1) * _CH, :],
                 preferred_element_type=jnp.float32) for t in range(_NCH)],
        axis=0)
    a = jnp.dot(mx.astype(jnp.bfloat16), w1,
                preferred_element_type=jnp.float32)
    h1 = jnp.maximum(a + b1, 0.0).astype(jnp.bfloat16)
    p2 = jnp.dot(h1, w2,
                 preferred_element_type=jnp.float32).astype(jnp.bfloat16)
    rows = _C * _S
    csums = []
    for k in range(_BPS):
        racc = jnp.zeros((_CH, _H), jnp.float32)
        for t in range(k * rows // _CH, (k + 1) * rows // _CH):
            m2c = jnp.dot(bd, p2[t * _CH:(t + 1) * _CH, :],
                          preferred_element_type=jnp.float32)
            racc = racc + jnp.maximum(m2c + b2, 0.0)
        csums.append(racc.sum(axis=0, keepdims=True))
    if _BPS < 8:
        csums.append(jnp.zeros((8 - _BPS, _H), jnp.float32))
    sums = jnp.concatenate(csums, axis=0)
    acc_ref[pl.ds(i * 8, 8), :] = sums * (1.0 / rows)

    @pl.when(i == _NSTEP - 1)
    def _head():
        p = jnp.concatenate(
            [acc_ref[k * 8:k * 8 + _BPS, :] for k in range(_NSTEP)], axis=0)
        h = jnp.maximum(
            jnp.dot(p, fc1w_ref[...], preferred_element_type=jnp.float32)
            + fc1b_ref[...], 0.0)
        out_ref[...] = (
            jnp.dot(h, fc2w_ref[...], preferred_element_type=jnp.float32)
            + fc2b_ref[...])


def kernel(x, W1, b1, W2, b2, fc1_W, fc1_b, fc2_W, fc2_b):
    xt = jnp.transpose(x, (0, 3, 1, 2)).reshape(_N, _F).astype(jnp.bfloat16)
    return pl.pallas_call(
        _body,
        grid=(_NSTEP,),
        in_specs=[
            pl.BlockSpec((_R, _F), lambda i: (i, 0)),
            pl.BlockSpec((_F, _H), lambda i: (0, 0)),
            pl.BlockSpec((1, _H), lambda i: (0, 0)),
            pl.BlockSpec((_H, _H), lambda i: (0, 0)),
            pl.BlockSpec((1, _H), lambda i: (0, 0)),
            pl.BlockSpec((_H, _H), lambda i: (0, 0)),
            pl.BlockSpec((1, _H), lambda i: (0, 0)),
            pl.BlockSpec((_H, _NS), lambda i: (0, 0)),
            pl.BlockSpec((1, _NS), lambda i: (0, 0)),
            pl.BlockSpec((_CH, _CH), lambda i: (0, 0)),
        ],
        out_specs=pl.BlockSpec((_B, _NS), lambda i: (0, 0)),
        out_shape=jax.ShapeDtypeStruct((_B, _NS), jnp.float32),
        scratch_shapes=[pltpu.VMEM((_NSTEP * 8, _H), jnp.float32)],
    )(xt, W1, b1.reshape(1, _H),
      W2, b2.reshape(1, _H),
      fc1_W, fc1_b.reshape(1, _H), fc2_W, fc2_b.reshape(1, _NS),
      jnp.asarray(_BD, jnp.bfloat16))
